# Initial kernel scaffold; baseline (speedup 1.0000x reference)
import jax
import jax.numpy as jnp
from jax.experimental import pallas as pl


def kernel(x, mask, inputP, W, b, ln1_g, ln1_b, ln2_g, ln2_b):
    raise NotImplementedError("write your pallas kernel here")


# speedup vs baseline: 3.2708x; 3.2708x over previous
import jax
import jax.numpy as jnp
from jax.experimental import pallas as pl

HIDDEN = 256
EPS = 1e-6


def _block_kernel(x_ref, p_ref, w_ref, b_ref, g1_ref, b1_ref, g2_ref, b2_ref,
                  o_ref):
    x = x_ref[0]
    p = p_ref[0]

    mu = jnp.mean(x, axis=-1, keepdims=True)
    xc = x - mu
    var = jnp.mean(xc * xc, axis=-1, keepdims=True)
    h = g1_ref[0] * xc / jnp.sqrt(var + EPS) + b1_ref[0]

    agg = jnp.dot(p, h, preferred_element_type=jnp.float32)
    conv = jnp.maximum(
        jnp.dot(agg, w_ref[...], preferred_element_type=jnp.float32)
        + b_ref[0], 0.0)

    y = x + conv
    mu2 = jnp.mean(y, axis=-1, keepdims=True)
    yc = y - mu2
    var2 = jnp.mean(yc * yc, axis=-1, keepdims=True)
    o_ref[0] = g2_ref[0] * yc / jnp.sqrt(var2 + EPS) + b2_ref[0]


def kernel(x, mask, inputP, W, b, ln1_g, ln1_b, ln2_g, ln2_b):
    del mask
    B, N, H = x.shape

    vec = lambda v: v.reshape(1, H)
    grid_spec = pl.GridSpec(
        grid=(B,),
        in_specs=[
            pl.BlockSpec((1, N, H), lambda i: (i, 0, 0)),
            pl.BlockSpec((1, N, N), lambda i: (i, 0, 0)),
            pl.BlockSpec((H, H), lambda i: (0, 0)),
            pl.BlockSpec((1, H), lambda i: (0, 0)),
            pl.BlockSpec((1, H), lambda i: (0, 0)),
            pl.BlockSpec((1, H), lambda i: (0, 0)),
            pl.BlockSpec((1, H), lambda i: (0, 0)),
            pl.BlockSpec((1, H), lambda i: (0, 0)),
        ],
        out_specs=pl.BlockSpec((1, N, H), lambda i: (i, 0, 0)),
    )
    return pl.pallas_call(
        _block_kernel,
        grid_spec=grid_spec,
        out_shape=jax.ShapeDtypeStruct((B, N, H), x.dtype),
    )(x, inputP, W, vec(b), vec(ln1_g), vec(ln1_b), vec(ln2_g), vec(ln2_b))
